# Initial kernel scaffold; baseline (speedup 1.0000x reference)
import jax
import jax.numpy as jnp
from jax.experimental import pallas as pl


def kernel(q, k, positive_indices, negative_indices):
    raise NotImplementedError("write your pallas kernel here")


# speedup vs baseline: 46.1004x; 46.1004x over previous
import functools

import jax
import jax.numpy as jnp
from jax import lax
from jax.experimental import pallas as pl
from jax.experimental.pallas import tpu as pltpu
from jax.experimental.pallas import tpu_sc as plsc

TEMP = 0.07
B, N, C, K = 8, 196, 384, 128
NPAD = 256
GROUPS = 129
GPAD = 144
NW = 32
NW_USED = 28
ROWS_PER_W = (B * N) // NW_USED
IDX_PER_W = ROWS_PER_W * GPAD


def _sim_body(q_ref, kp_ref, s_ref):
    qb = q_ref[0]
    kb = kp_ref[0]
    qn = qb / jnp.maximum(jnp.sqrt(jnp.sum(qb * qb, axis=-1, keepdims=True)), 1e-12)
    kn = kb / jnp.maximum(jnp.sqrt(jnp.sum(kb * kb, axis=-1, keepdims=True)), 1e-12)
    s = lax.dot_general(qn, kn, (((1,), (1,)), ((), ())),
                        preferred_element_type=jnp.float32)
    s_ref[0] = s / TEMP


def _loss_body(x_ref, o_ref):
    x = x_ref[:]
    lane = lax.broadcasted_iota(jnp.int32, x.shape, 1)
    valid = lane < GROUPS
    m = jnp.max(jnp.where(valid, x, -1e30), axis=1, keepdims=True)
    e = jnp.where(valid, jnp.exp(x - m), 0.0)
    lse = m + jnp.log(jnp.sum(e, axis=1, keepdims=True))
    per_row = lse - x[:, 0:1]
    o_ref[:, :] = (jnp.sum(per_row) / (B * N)).reshape(1, 1)


def _gather_body(s_hbm, idx_hbm, out_hbm, s_v, idx_v, out_v):
    nc = plsc.get_sparse_core_info().num_cores
    wid = lax.axis_index("s") * nc + lax.axis_index("c")

    @pl.when(wid < NW_USED)
    def _():
        pltpu.sync_copy(s_hbm.at[pl.ds(wid * (ROWS_PER_W * NPAD), ROWS_PER_W * NPAD)],
                        s_v)
        pltpu.sync_copy(idx_hbm.at[pl.ds(wid * IDX_PER_W, IDX_PER_W)], idx_v)

        def row_body(r, carry):
            base = r * NPAD

            def grp_body(g, c2):
                off = r * GPAD + g * 16
                cols = idx_v[pl.ds(off, 16)] + base
                out_v[pl.ds(off, 16)] = plsc.load_gather(s_v, [cols])
                return c2

            return lax.fori_loop(0, GPAD // 16, grp_body, carry)

        lax.fori_loop(0, ROWS_PER_W, row_body, 0)
        pltpu.sync_copy(out_v, out_hbm.at[pl.ds(wid * IDX_PER_W, IDX_PER_W)])


@functools.cache
def _gather_call():
    return pl.kernel(
        _gather_body,
        mesh=plsc.VectorSubcoreMesh(core_axis_name="c", subcore_axis_name="s"),
        out_type=jax.ShapeDtypeStruct((NW_USED * IDX_PER_W,), jnp.float32),
        scratch_types=[
            pltpu.VMEM((ROWS_PER_W * NPAD,), jnp.float32),
            pltpu.VMEM((IDX_PER_W,), jnp.int32),
            pltpu.VMEM((IDX_PER_W,), jnp.float32),
        ],
        compiler_params=pltpu.CompilerParams(needs_layout_passes=False),
    )


def kernel(q, k, positive_indices, negative_indices):
    k_pad = jnp.pad(k, ((0, 0), (0, NPAD - N), (0, 0)))
    s = pl.pallas_call(
        _sim_body,
        grid=(B,),
        in_specs=[
            pl.BlockSpec((1, N, C), lambda b: (b, 0, 0)),
            pl.BlockSpec((1, NPAD, C), lambda b: (b, 0, 0)),
        ],
        out_specs=pl.BlockSpec((1, N, NPAD), lambda b: (b, 0, 0)),
        out_shape=jax.ShapeDtypeStruct((B, N, NPAD), jnp.float32),
    )(q, k_pad)
    s_flat = s.reshape(B * N * NPAD)

    idx = jnp.concatenate(
        [positive_indices[..., None], negative_indices], axis=2
    ).astype(jnp.int32)
    idx = jnp.pad(idx, ((0, 0), (0, 0), (0, GPAD - GROUPS)))
    idx = idx.reshape(NW_USED * IDX_PER_W)

    logits = _gather_call()(s_flat, idx).reshape(B * N, GPAD)

    loss = pl.pallas_call(
        _loss_body,
        out_shape=jax.ShapeDtypeStruct((1, 1), jnp.float32),
    )(logits)
    return loss[0, 0]


# speedup vs baseline: 47.8591x; 1.0381x over previous
import functools

import jax
import jax.numpy as jnp
from jax import lax
from jax.experimental import pallas as pl
from jax.experimental.pallas import tpu as pltpu
from jax.experimental.pallas import tpu_sc as plsc

TEMP = 0.07
B, N, C, K = 8, 196, 384, 128
NPAD = 256
GPAD = 144
NW = 32
NW_USED = 28
ROWS_PER_W = (B * N) // NW_USED
IDX_PER_W = ROWS_PER_W * GPAD
ROW_BLKS = (ROWS_PER_W + 15) // 16


def _sim_body(q_ref, kp_ref, s_ref):
    qb = q_ref[0]
    kb = kp_ref[0]
    qn = qb / jnp.maximum(jnp.sqrt(jnp.sum(qb * qb, axis=-1, keepdims=True)), 1e-12)
    kn = kb / jnp.maximum(jnp.sqrt(jnp.sum(kb * kb, axis=-1, keepdims=True)), 1e-12)
    s = lax.dot_general(qn, kn, (((1,), (1,)), ((), ())),
                        preferred_element_type=jnp.float32)
    s_ref[0] = s / TEMP


def _loss_body(x_ref, o_ref):
    x = x_ref[:]
    lane = lax.broadcasted_iota(jnp.int32, x.shape, 1)
    valid = lane <= K
    m = jnp.max(jnp.where(valid, x, -1e30), axis=1, keepdims=True)
    e = jnp.where(valid, jnp.exp(x - m), 0.0)
    lse = m + jnp.log(jnp.sum(e, axis=1, keepdims=True))
    per_row = lse - x[:, K:K + 1]
    o_ref[:, :] = (jnp.sum(per_row) / (B * N)).reshape(1, 1)


def _gather_body(s_hbm, pos_hbm, neg_hbm, out_hbm, s_v, pos_v, neg_v, out_v):
    nc = plsc.get_sparse_core_info().num_cores
    wid = lax.axis_index("s") * nc + lax.axis_index("c")

    @pl.when(wid < NW_USED)
    def _():
        pltpu.sync_copy(s_hbm.at[pl.ds(wid * (ROWS_PER_W * NPAD), ROWS_PER_W * NPAD)],
                        s_v)
        pltpu.sync_copy(neg_hbm.at[pl.ds(wid * (ROWS_PER_W * K), ROWS_PER_W * K)],
                        neg_v)
        pos_v[pl.ds(16 * (ROW_BLKS - 1), 16)] = jnp.zeros((16,), jnp.int32)
        pltpu.sync_copy(pos_hbm.at[pl.ds(wid * ROWS_PER_W, ROWS_PER_W)],
                        pos_v.at[pl.ds(0, ROWS_PER_W)])

        def row_body(r, carry):
            base = r * NPAD
            for g in range(K // 16):
                cols = neg_v[pl.ds(r * K + g * 16, 16)] + base
                out_v[pl.ds(r * GPAD + g * 16, 16)] = plsc.load_gather(s_v, [cols])
            return carry

        lax.fori_loop(0, ROWS_PER_W, row_body, 0)

        lanes = lax.iota(jnp.int32, 16)
        for t in range(ROW_BLKS):
            rows = lanes + t * 16
            valid = rows < ROWS_PER_W
            src = jnp.where(valid, rows * NPAD + pos_v[pl.ds(t * 16, 16)], 0)
            pvals = plsc.load_gather(s_v, [src])
            plsc.store_scatter(out_v, [rows * GPAD + K], pvals, mask=valid)

        pltpu.sync_copy(out_v, out_hbm.at[pl.ds(wid * IDX_PER_W, IDX_PER_W)])


@functools.cache
def _gather_call():
    return pl.kernel(
        _gather_body,
        mesh=plsc.VectorSubcoreMesh(core_axis_name="c", subcore_axis_name="s"),
        out_type=jax.ShapeDtypeStruct((NW_USED * IDX_PER_W,), jnp.float32),
        scratch_types=[
            pltpu.VMEM((ROWS_PER_W * NPAD,), jnp.float32),
            pltpu.VMEM((16 * ROW_BLKS,), jnp.int32),
            pltpu.VMEM((ROWS_PER_W * K,), jnp.int32),
            pltpu.VMEM((IDX_PER_W,), jnp.float32),
        ],
        compiler_params=pltpu.CompilerParams(needs_layout_passes=False),
    )


def kernel(q, k, positive_indices, negative_indices):
    k_pad = jnp.pad(k, ((0, 0), (0, NPAD - N), (0, 0)))
    s = pl.pallas_call(
        _sim_body,
        grid=(B,),
        in_specs=[
            pl.BlockSpec((1, N, C), lambda b: (b, 0, 0)),
            pl.BlockSpec((1, NPAD, C), lambda b: (b, 0, 0)),
        ],
        out_specs=pl.BlockSpec((1, N, NPAD), lambda b: (b, 0, 0)),
        out_shape=jax.ShapeDtypeStruct((B, N, NPAD), jnp.float32),
    )(q, k_pad)
    s_flat = s.reshape(B * N * NPAD)

    pos = positive_indices.astype(jnp.int32).reshape(B * N)
    neg = negative_indices.astype(jnp.int32).reshape(B * N * K)

    logits = _gather_call()(s_flat, pos, neg).reshape(B * N, GPAD)

    loss = pl.pallas_call(
        _loss_body,
        out_shape=jax.ShapeDtypeStruct((1, 1), jnp.float32),
    )(logits)
    return loss[0, 0]
